# baseline (speedup 1.0000x reference)
import functools

import jax
import jax.numpy as jnp
from jax import lax
from jax.experimental import pallas as pl
from jax.experimental.pallas import tpu as pltpu
from jax.experimental.pallas import tpu_sc as plsc

_MAX_LEN = 4194304
_N_APPENDS = 6291456
_H = _N_APPENDS - _MAX_LEN
_NW = 32
_PER_W = _MAX_LEN // _NW


_CHUNK = 32768
_NCHUNK = _PER_W // _CHUNK


@functools.partial(
    pl.kernel,
    mesh=plsc.VectorSubcoreMesh(core_axis_name="c", subcore_axis_name="s"),
    out_type=jax.ShapeDtypeStruct((_MAX_LEN,), jnp.float32),
    scratch_types=[
        pltpu.VMEM((2, _CHUNK), jnp.float32),
        pltpu.SemaphoreType.DMA,
        pltpu.SemaphoreType.DMA,
        pltpu.SemaphoreType.DMA,
        pltpu.SemaphoreType.DMA,
    ],
)
def _ring_rotate(values_hbm, out_hbm, buf, sem_in0, sem_in1, sem_out0, sem_out1):
    sem_in = (sem_in0, sem_in1)
    sem_out = (sem_out0, sem_out1)
    wid = lax.axis_index("s") * 2 + lax.axis_index("c")
    dst = wid * _PER_W
    src = dst + jnp.where(dst < _H, _MAX_LEN, 0)

    def rd(k):
        return pltpu.make_async_copy(
            values_hbm.at[pl.ds(src + k * _CHUNK, _CHUNK)], buf.at[k % 2],
            sem_in[k % 2])

    def wr(k):
        return pltpu.make_async_copy(
            buf.at[k % 2], out_hbm.at[pl.ds(dst + k * _CHUNK, _CHUNK)],
            sem_out[k % 2])

    rd(0).start()
    for k in range(_NCHUNK):
        rd(k).wait()
        if k + 1 < _NCHUNK:
            if k >= 1:
                wr(k - 1).wait()
            rd(k + 1).start()
        wr(k).start()
    wr(_NCHUNK - 2).wait()
    wr(_NCHUNK - 1).wait()


def kernel(values, buffer):
    del buffer
    return _ring_rotate(values)


# speedup vs baseline: 1.0103x; 1.0103x over previous
import functools

import jax
import jax.numpy as jnp
from jax import lax
from jax.experimental import pallas as pl
from jax.experimental.pallas import tpu as pltpu
from jax.experimental.pallas import tpu_sc as plsc

_MAX_LEN = 4194304
_N_APPENDS = 6291456
_H = _N_APPENDS - _MAX_LEN
_NW = 32
_PER_W = _MAX_LEN // _NW


_CHUNK = 16384
_NCHUNK = _PER_W // _CHUNK
_NSLOT = 4


@functools.partial(
    pl.kernel,
    mesh=plsc.VectorSubcoreMesh(core_axis_name="c", subcore_axis_name="s"),
    out_type=jax.ShapeDtypeStruct((_MAX_LEN,), jnp.float32),
    scratch_types=[pltpu.VMEM((_NSLOT, _CHUNK), jnp.float32)]
    + [pltpu.SemaphoreType.DMA] * (2 * _NSLOT),
)
def _ring_rotate(values_hbm, out_hbm, buf, *sems):
    sem_in = sems[:_NSLOT]
    sem_out = sems[_NSLOT:]
    wid = lax.axis_index("s") * 2 + lax.axis_index("c")
    dst = wid * _PER_W
    src = dst + jnp.where(dst < _H, _MAX_LEN, 0)

    def rd(k):
        return pltpu.make_async_copy(
            values_hbm.at[pl.ds(src + k * _CHUNK, _CHUNK)], buf.at[k % _NSLOT],
            sem_in[k % _NSLOT])

    def wr(k):
        return pltpu.make_async_copy(
            buf.at[k % _NSLOT], out_hbm.at[pl.ds(dst + k * _CHUNK, _CHUNK)],
            sem_out[k % _NSLOT])

    rd(0).start()
    rd(1).start()
    for k in range(_NCHUNK):
        rd(k).wait()
        wr(k).start()
        if k + 2 < _NCHUNK:
            if k >= 2:
                wr(k - 2).wait()
            rd(k + 2).start()
    for k in range(_NCHUNK - _NSLOT, _NCHUNK):
        wr(k).wait()


def kernel(values, buffer):
    del buffer
    return _ring_rotate(values)
